# baseline (speedup 1.0000x reference)
import functools

import jax
import jax.numpy as jnp
from jax import lax
from jax.experimental import pallas as pl
from jax.experimental.pallas import tpu as pltpu
from jax.experimental.pallas import tpu_sc as plsc

B = 16384
V = 1000000
D = 64
C = 4

NC = 2
NS = 16
NW = NC * NS
BPW = B // NW
CH = 128
NCHUNK = BPW // CH


def _sto_body(idx_hbm, ci_hbm, eps_hbm, w_hbm, comb_hbm, out_hbm,
              idx_v, ci_v, emb_v, comb_v, eps_v, sem_w, sem_c):
    wid = lax.axis_index("s") * NC + lax.axis_index("c")
    base = wid * BPW

    for k in range(NCHUNK):
        off = base + k * CH
        pltpu.sync_copy(idx_hbm.at[pl.ds(off, CH)], idx_v)
        pltpu.sync_copy(ci_hbm.at[pl.ds(off, CH)], ci_v)
        gw = pltpu.async_copy(w_hbm.at[idx_v], emb_v, sem_w)
        gc = pltpu.async_copy(comb_hbm.at[ci_v], comb_v, sem_c)
        pltpu.sync_copy(eps_hbm.at[pl.ds(off, CH)], eps_v)
        gw.wait()
        gc.wait()

        def row_body(r, _):
            for c in range(D // 16):
                sl = pl.ds(c * 16, 16)
                w = emb_v[r, sl]
                e = eps_v[r, sl]
                m = comb_v[r, pl.ds(c * 16, 16)]
                s = comb_v[r, pl.ds(D + c * 16, 16)]
                emb_v[r, sl] = w * (m + s * e)
            return 0

        lax.fori_loop(0, CH, row_body, 0)
        pltpu.sync_copy(emb_v, out_hbm.at[pl.ds(off, CH)])


@functools.partial(jax.jit, static_argnames=())
def _sto(indices, comp_indices, epsilon, weight, comb):
    mesh = plsc.VectorSubcoreMesh(core_axis_name="c", subcore_axis_name="s")
    run = pl.kernel(
        _sto_body,
        out_type=jax.ShapeDtypeStruct((B, D), jnp.float32),
        mesh=mesh,
        compiler_params=pltpu.CompilerParams(use_tc_tiling_on_sc=False),
        scratch_types=[
            pltpu.VMEM((CH,), jnp.int32),
            pltpu.VMEM((CH,), jnp.int32),
            pltpu.VMEM((CH, D), jnp.float32),
            pltpu.VMEM((CH, 2 * D), jnp.float32),
            pltpu.VMEM((CH, D), jnp.float32),
            pltpu.SemaphoreType.DMA,
            pltpu.SemaphoreType.DMA,
        ],
    )
    return run(indices, comp_indices, epsilon, weight, comb)


def kernel(indices, comp_indices, epsilon, weight, post_mean, post_std):
    comb = jnp.concatenate(
        [post_mean, jax.nn.softplus(post_std)], axis=1).astype(jnp.float32)
    return _sto(indices.astype(jnp.int32), comp_indices.astype(jnp.int32),
                epsilon, weight, comb)


# speedup vs baseline: 2.0288x; 2.0288x over previous
import functools

import jax
import jax.numpy as jnp
from jax import lax
from jax.experimental import pallas as pl
from jax.experimental.pallas import tpu as pltpu
from jax.experimental.pallas import tpu_sc as plsc

B = 16384
V = 1000000
D = 64
C = 4

NC = 2
NS = 16
NW = NC * NS
BPW = B // NW
CH = 256
NCHUNK = BPW // CH


def _sto_body(idx_hbm, ci_hbm, eps_hbm, w_hbm, comb_hbm, out_hbm,
              idx_v, ci_v, emb_v, eps_v, comb_v, sem_w, sem_e):
    wid = lax.axis_index("s") * NC + lax.axis_index("c")
    base = wid * BPW

    pltpu.sync_copy(comb_hbm, comb_v)

    for k in range(NCHUNK):
        off = base + k * CH
        pltpu.sync_copy(idx_hbm.at[pl.ds(off, CH)], idx_v)
        pltpu.sync_copy(ci_hbm.at[pl.ds(off, CH)], ci_v)

        def fire(g, _):
            ivec = idx_v[pl.ds(g * 16, 16)]
            for j in range(16):
                pltpu.async_copy(w_hbm.at[ivec[j]], emb_v.at[g * 16 + j],
                                 sem_w)
            return 0

        lax.fori_loop(0, CH // 16, fire, 0)
        ge = pltpu.async_copy(eps_hbm.at[pl.ds(off, CH)], eps_v, sem_e)
        pltpu.make_async_copy(eps_hbm.at[pl.ds(off, CH)], emb_v, sem_w).wait()
        ge.wait()

        def row_body(g, _):
            cvec = ci_v[pl.ds(g * 16, 16)]
            for j in range(16):
                r = g * 16 + j
                c = cvec[j]
                for q in range(D // 16):
                    sl = pl.ds(q * 16, 16)
                    w = emb_v[r, sl]
                    e = eps_v[r, sl]
                    m = comb_v[c, pl.ds(q * 16, 16)]
                    s = comb_v[c, pl.ds(D + q * 16, 16)]
                    emb_v[r, sl] = w * (m + s * e)
            return 0

        lax.fori_loop(0, CH // 16, row_body, 0)
        pltpu.sync_copy(emb_v, out_hbm.at[pl.ds(off, CH)])


@jax.jit
def _sto(indices, comp_indices, epsilon, weight, comb):
    mesh = plsc.VectorSubcoreMesh(core_axis_name="c", subcore_axis_name="s")
    run = pl.kernel(
        _sto_body,
        out_type=jax.ShapeDtypeStruct((B, D), jnp.float32),
        mesh=mesh,
        compiler_params=pltpu.CompilerParams(use_tc_tiling_on_sc=True),
        scratch_types=[
            pltpu.VMEM((CH,), jnp.int32),
            pltpu.VMEM((CH,), jnp.int32),
            pltpu.VMEM((CH, D), jnp.float32),
            pltpu.VMEM((CH, D), jnp.float32),
            pltpu.VMEM((C, 2 * D), jnp.float32),
            pltpu.SemaphoreType.DMA,
            pltpu.SemaphoreType.DMA,
        ],
    )
    return run(indices, comp_indices, epsilon, weight, comb)


def kernel(indices, comp_indices, epsilon, weight, post_mean, post_std):
    comb = jnp.concatenate(
        [post_mean, jax.nn.softplus(post_std)], axis=1).astype(jnp.float32)
    return _sto(indices.astype(jnp.int32), comp_indices.astype(jnp.int32),
                epsilon, weight, comb)


# speedup vs baseline: 2.9731x; 1.4654x over previous
import jax
import jax.numpy as jnp
from jax import lax
from jax.experimental import pallas as pl
from jax.experimental.pallas import tpu as pltpu
from jax.experimental.pallas import tpu_sc as plsc

B = 16384
V = 1000000
D = 64
C = 4

NC = 2
NS = 16
NW = NC * NS
BPW = B // NW
CH = 256
NCHUNK = BPW // CH


def _sto_body(idx_hbm, ci_hbm, eps_hbm, w3_hbm, comb_hbm, out_hbm,
              idx_v, ci_v, emb_v, eps_v, comb_v, sem_w, sem_e, sem_c):
    wid = lax.axis_index("s") * NC + lax.axis_index("c")
    base = wid * BPW

    gc = pltpu.async_copy(comb_hbm, comb_v, sem_c)

    for k in range(NCHUNK):
        off = base + k * CH
        pltpu.sync_copy(idx_hbm.at[pl.ds(off, CH)], idx_v)
        pltpu.sync_copy(ci_hbm.at[pl.ds(off, CH)], ci_v)

        def fire(g, _):
            ivec = idx_v[pl.ds(g * 16, 16)]
            for j in range(16):
                i = ivec[j]
                pltpu.async_copy(w3_hbm.at[i >> 3, i & 7],
                                 emb_v.at[g * 16 + j], sem_w)
            return 0

        lax.fori_loop(0, CH // 16, fire, 0)
        ge = pltpu.async_copy(eps_hbm.at[pl.ds(off, CH)], eps_v, sem_e)
        pltpu.make_async_copy(eps_hbm.at[pl.ds(off, CH)], emb_v, sem_w).wait()
        ge.wait()
        if k == 0:
            gc.wait()

        def row_body(g, _):
            cvec = ci_v[pl.ds(g * 16, 16)]
            for j in range(16):
                r = g * 16 + j
                c = cvec[j]
                for q in range(D // 16):
                    sl = pl.ds(q * 16, 16)
                    w = emb_v[r, sl]
                    e = eps_v[r, sl]
                    m = comb_v[c, pl.ds(q * 16, 16)]
                    s = comb_v[c, pl.ds(D + q * 16, 16)]
                    emb_v[r, sl] = w * (m + s * e)
            return 0

        lax.fori_loop(0, CH // 16, row_body, 0)
        pltpu.sync_copy(emb_v, out_hbm.at[pl.ds(off, CH)])


@jax.jit
def _sto(indices, comp_indices, epsilon, w3, comb):
    mesh = plsc.VectorSubcoreMesh(core_axis_name="c", subcore_axis_name="s")
    run = pl.kernel(
        _sto_body,
        out_type=jax.ShapeDtypeStruct((B, D), jnp.float32),
        mesh=mesh,
        compiler_params=pltpu.CompilerParams(use_tc_tiling_on_sc=True),
        scratch_types=[
            pltpu.VMEM((CH,), jnp.int32),
            pltpu.VMEM((CH,), jnp.int32),
            pltpu.VMEM((CH, D), jnp.float32),
            pltpu.VMEM((CH, D), jnp.float32),
            pltpu.VMEM((C, 2 * D), jnp.float32),
            pltpu.SemaphoreType.DMA,
            pltpu.SemaphoreType.DMA,
            pltpu.SemaphoreType.DMA,
        ],
    )
    return run(indices, comp_indices, epsilon, w3, comb)


def kernel(indices, comp_indices, epsilon, weight, post_mean, post_std):
    comb = jnp.concatenate(
        [post_mean, jax.nn.softplus(post_std)], axis=1).astype(jnp.float32)
    w3 = weight.reshape(V // 8, 8, D)
    return _sto(indices.astype(jnp.int32), comp_indices.astype(jnp.int32),
                epsilon, w3, comb)


# speedup vs baseline: 4.3366x; 1.4586x over previous
import jax
import jax.numpy as jnp
from jax import lax
from jax.experimental import pallas as pl
from jax.experimental.pallas import tpu as pltpu
from jax.experimental.pallas import tpu_sc as plsc

B = 16384
V = 1000000
D = 64
C = 4

NC = 2
NS = 16
NW = NC * NS
BPW = B // NW
WV = 4
NWAVE = BPW // WV
RING = 8
HB = 256

_IOTA = None


def _sto_body(p1_hbm, p2_hbm, eps_hbm, wT_hbm, comb_hbm, out_hbm,
              p1_v, p2_v, eps_v, ring_v, comb_v,
              sem_e, sem_w0, sem_w1, sem_o, sem_c):
    wid = lax.axis_index("s") * NC + lax.axis_index("c")
    base = wid * BPW
    iota = lax.iota(jnp.int32, 16)

    gc = pltpu.async_copy(comb_hbm, comb_v, sem_c)
    pltpu.sync_copy(p1_hbm.at[pl.ds(base, BPW + 16)], p1_v)
    pltpu.sync_copy(p2_hbm.at[pl.ds(base, BPW + 16)], p2_v)

    def fetch_lane(p, sem):
        fl = (p >> 16) & 1

        @pl.when(fl == 1)
        def _():
            off = pl.multiple_of((p & 0x1FFF) * 128, 128)
            rs = (p >> 13) & 7
            pltpu.async_copy(wT_hbm.at[:, pl.ds(off, 128)], ring_v.at[rs],
                             sem)

    def drain_lane(p, sem):
        fl = (p >> 16) & 1

        @pl.when(fl == 1)
        def _():
            pltpu.make_async_copy(wT_hbm.at[:, pl.ds(0, 128)],
                                  ring_v.at[0], sem).wait()

    def wave(v, _):
        p1v = p1_v[pl.ds(v * WV, 16)]
        p2v = p2_v[pl.ds(v * WV, 16)]

        @pl.when(v < NWAVE - 1)
        def _():
            @pl.when(v % 2 == 0)
            def _():
                for l in range(WV, 2 * WV):
                    fetch_lane(p1v[l], sem_w1)

            @pl.when(v % 2 == 1)
            def _():
                for l in range(WV, 2 * WV):
                    fetch_lane(p1v[l], sem_w0)

        @pl.when(v % 2 == 0)
        def _():
            for l in range(WV):
                drain_lane(p1v[l], sem_w0)

        @pl.when(v % 2 == 1)
        def _():
            for l in range(WV):
                drain_lane(p1v[l], sem_w1)

        for l in range(WV):
            p = p1v[l]
            rs = (p >> 13) & 7
            col = (p >> 17) & 0x7F
            c = (p >> 24) & 3
            so = p2v[l]
            r = v * WV + l
            er = r & (HB - 1)
            for q in range(D // 16):
                sl = pl.ds(q * 16, 16)
                w = plsc.load_gather(
                    ring_v, [jnp.full((16,), rs, jnp.int32),
                             q * 16 + iota,
                             jnp.full((16,), col, jnp.int32)])
                e = eps_v[er, sl]
                m = comb_v[c, sl]
                s = comb_v[c, pl.ds(D + q * 16, 16)]
                eps_v[er, sl] = w * (m + s * e)
            pltpu.async_copy(eps_v.at[er], out_hbm.at[so], sem_o)
        return 0

    for h in range(BPW // HB):
        def eps_fire(g, _):
            ivec = p2_v[pl.ds(h * HB + g * 16, 16)]
            for j in range(16):
                pltpu.async_copy(eps_hbm.at[ivec[j]], eps_v.at[g * 16 + j],
                                 sem_e)
            return 0

        if h > 0:
            pltpu.make_async_copy(eps_hbm.at[pl.ds(0, HB)], eps_v,
                                  sem_o).wait()
        lax.fori_loop(0, HB // 16, eps_fire, 0)
        if h == 0:
            p1v0 = p1_v[pl.ds(0, 16)]
            for l in range(WV):
                fetch_lane(p1v0[l], sem_w0)
        pltpu.make_async_copy(eps_hbm.at[pl.ds(0, HB)], eps_v, sem_e).wait()
        if h == 0:
            gc.wait()
        lax.fori_loop(h * (HB // WV), (h + 1) * (HB // WV), wave, 0)

    pltpu.make_async_copy(eps_hbm.at[pl.ds(0, HB)], eps_v, sem_o).wait()


@jax.jit
def _sto(p1, p2, epsilon, weightT, comb):
    mesh = plsc.VectorSubcoreMesh(core_axis_name="c", subcore_axis_name="s")
    run = pl.kernel(
        _sto_body,
        out_type=jax.ShapeDtypeStruct((B, D), jnp.float32),
        mesh=mesh,
        compiler_params=pltpu.CompilerParams(
            use_tc_tiling_on_sc=True, disable_bounds_checks=True,
            needs_layout_passes=False),
        scratch_types=[
            pltpu.VMEM((BPW + 16,), jnp.int32),
            pltpu.VMEM((BPW + 16,), jnp.int32),
            pltpu.VMEM((HB, D), jnp.float32),
            pltpu.VMEM((RING, D, 128), jnp.float32),
            pltpu.VMEM((C, 2 * D), jnp.float32),
            pltpu.SemaphoreType.DMA,
            pltpu.SemaphoreType.DMA,
            pltpu.SemaphoreType.DMA,
            pltpu.SemaphoreType.DMA,
            pltpu.SemaphoreType.DMA,
        ],
    )
    return run(p1, p2, epsilon, weightT, comb)


def kernel(indices, comp_indices, epsilon, weight, post_mean, post_std):
    comb = jnp.concatenate(
        [post_mean, jax.nn.softplus(post_std)], axis=1).astype(jnp.float32)
    idx = indices.astype(jnp.int32)
    ci = comp_indices.astype(jnp.int32)
    blk = idx >> 7
    col = idx & 127
    order = jnp.argsort(blk).astype(jnp.int32)
    sblk = blk[order]
    scol = col[order]
    sci = ci[order]
    pos = jnp.arange(B, dtype=jnp.int32)
    first = jnp.concatenate(
        [jnp.ones((1,), jnp.int32),
         (sblk[1:] != sblk[:-1]).astype(jnp.int32)])
    flag = jnp.maximum(first, (pos % BPW == 0).astype(jnp.int32))
    rs = (jnp.cumsum(flag) - 1).astype(jnp.int32) & 7
    p1 = sblk | (rs << 13) | (flag << 16) | (scol << 17) | (sci << 24)
    p2 = order
    p1 = jnp.pad(p1, (0, 16))
    p2 = jnp.pad(p2, (0, 16))
    return _sto(p1, p2, epsilon, weight.T, comb)


# speedup vs baseline: 5.2370x; 1.2076x over previous
import jax
import jax.numpy as jnp
from jax import lax
from jax.experimental import pallas as pl
from jax.experimental.pallas import tpu as pltpu
from jax.experimental.pallas import tpu_sc as plsc

B = 16384
V = 1000000
D = 64
C = 4

NC = 2
NS = 16
NW = NC * NS
BPW = B // NW
WV = 2
NWAVE = BPW // WV
DEPTH = 3
RING = 10
HB = 256


def _sto_body(p1_hbm, p2_hbm, eps_hbm, wT_hbm, comb_hbm, out_hbm,
              p1_v, p2_v, eps_v, ring_v, comb_v,
              sem_e, sem_o, sem_c, *sem_w):
    wid = lax.axis_index("s") * NC + lax.axis_index("c")
    base = wid * BPW
    iota = lax.iota(jnp.int32, 16)

    gc = pltpu.async_copy(comb_hbm, comb_v, sem_c)
    pltpu.sync_copy(p1_hbm.at[pl.ds(base, BPW + 16)], p1_v)
    pltpu.sync_copy(p2_hbm.at[pl.ds(base, BPW + 16)], p2_v)

    def fetch_lane(p, sem):
        fl = (p >> 17) & 1

        @pl.when(fl == 1)
        def _():
            off = pl.multiple_of((p & 0x1FFF) * 128, 128)
            rs = (p >> 13) & 15
            pltpu.async_copy(wT_hbm.at[:, pl.ds(off, 128)], ring_v.at[rs],
                             sem)

    def drain_lane(p, sem):
        fl = (p >> 17) & 1

        @pl.when(fl == 1)
        def _():
            pltpu.make_async_copy(wT_hbm.at[:, pl.ds(0, 128)],
                                  ring_v.at[0], sem).wait()

    def wave(v, _):
        p1v = p1_v[pl.ds(v * WV, 16)]
        p2v = p2_v[pl.ds(v * WV, 16)]

        for par in range(DEPTH):
            @pl.when(v % DEPTH == par)
            def _(par=par):
                @pl.when(v < NWAVE - DEPTH)
                def _():
                    for l in range(DEPTH * WV, (DEPTH + 1) * WV):
                        fetch_lane(p1v[l], sem_w[par])
                for l in range(WV):
                    drain_lane(p1v[l], sem_w[par])

        for l in range(WV):
            p = p1v[l]
            rs = (p >> 13) & 15
            col = (p >> 18) & 0x7F
            c = (p >> 25) & 3
            so = p2v[l]
            r = v * WV + l
            er = r & (HB - 1)
            for q in range(D // 16):
                sl = pl.ds(q * 16, 16)
                w = plsc.load_gather(
                    ring_v, [jnp.full((16,), rs, jnp.int32),
                             q * 16 + iota,
                             jnp.full((16,), col, jnp.int32)])
                e = eps_v[er, sl]
                m = comb_v[c, sl]
                s = comb_v[c, pl.ds(D + q * 16, 16)]
                eps_v[er, sl] = w * (m + s * e)
            pltpu.async_copy(eps_v.at[er], out_hbm.at[so], sem_o)
        return 0

    for h in range(BPW // HB):
        def eps_fire(g, _):
            ivec = p2_v[pl.ds(h * HB + g * 16, 16)]
            for j in range(16):
                pltpu.async_copy(eps_hbm.at[ivec[j]], eps_v.at[g * 16 + j],
                                 sem_e)
            return 0

        if h > 0:
            pltpu.make_async_copy(eps_hbm.at[pl.ds(0, HB)], eps_v,
                                  sem_o).wait()
        lax.fori_loop(0, HB // 16, eps_fire, 0)
        if h == 0:
            p1v0 = p1_v[pl.ds(0, 16)]
            for v0 in range(DEPTH):
                for l in range(WV):
                    fetch_lane(p1v0[v0 * WV + l], sem_w[v0 % DEPTH])
        pltpu.make_async_copy(eps_hbm.at[pl.ds(0, HB)], eps_v, sem_e).wait()
        if h == 0:
            gc.wait()
        lax.fori_loop(h * (HB // WV), (h + 1) * (HB // WV), wave, 0)

    pltpu.make_async_copy(eps_hbm.at[pl.ds(0, HB)], eps_v, sem_o).wait()


@jax.jit
def _sto(p1, p2, epsilon, weightT, comb):
    mesh = plsc.VectorSubcoreMesh(core_axis_name="c", subcore_axis_name="s")
    run = pl.kernel(
        _sto_body,
        out_type=jax.ShapeDtypeStruct((B, D), jnp.float32),
        mesh=mesh,
        compiler_params=pltpu.CompilerParams(
            use_tc_tiling_on_sc=True, disable_bounds_checks=True,
            needs_layout_passes=False),
        scratch_types=[
            pltpu.VMEM((BPW + 16,), jnp.int32),
            pltpu.VMEM((BPW + 16,), jnp.int32),
            pltpu.VMEM((HB, D), jnp.float32),
            pltpu.VMEM((RING, D, 128), jnp.float32),
            pltpu.VMEM((C, 2 * D), jnp.float32),
            pltpu.SemaphoreType.DMA,
            pltpu.SemaphoreType.DMA,
            pltpu.SemaphoreType.DMA,
        ] + [pltpu.SemaphoreType.DMA] * DEPTH,
    )
    return run(p1, p2, epsilon, weightT, comb)


def kernel(indices, comp_indices, epsilon, weight, post_mean, post_std):
    comb = jnp.concatenate(
        [post_mean, jax.nn.softplus(post_std)], axis=1).astype(jnp.float32)
    idx = indices.astype(jnp.int32)
    ci = comp_indices.astype(jnp.int32)
    blk = idx >> 7
    col = idx & 127
    orig = jnp.arange(B, dtype=jnp.int32)
    payload = (col << 18) | (ci << 16) | orig
    sblk, sp = lax.sort((blk, payload), num_keys=1)
    pos = jnp.arange(B, dtype=jnp.int32)
    first = jnp.concatenate(
        [jnp.ones((1,), jnp.int32),
         (sblk[1:] != sblk[:-1]).astype(jnp.int32)])
    flag = jnp.maximum(first, (pos % BPW == 0).astype(jnp.int32))
    rs = (jnp.cumsum(flag) - 1).astype(jnp.int32) % RING
    p1 = (sblk | (rs << 13) | (flag << 17) | (sp & (0x7F << 18))
          | (((sp >> 16) & 3) << 25))
    p2 = sp & 0x3FFF
    p1 = jnp.pad(p1, (0, 16))
    p2 = jnp.pad(p2, (0, 16))
    return _sto(p1, p2, epsilon, weight.T, comb)


# speedup vs baseline: 5.5560x; 1.0609x over previous
import jax
import jax.numpy as jnp
from jax import lax
from jax.experimental import pallas as pl
from jax.experimental.pallas import tpu as pltpu
from jax.experimental.pallas import tpu_sc as plsc

B = 16384
V = 1000000
D = 64
C = 4

NC = 2
NS = 16
NW = NC * NS
BPW = B // NW
WV = 2
NWAVE = BPW // WV
DEPTH = 4
RING = 11
HB = 128


def _sto_body(p1_hbm, p2_hbm, eps_hbm, wT_hbm, comb_hbm, out_hbm,
              p1_v, p2_v, eps_v, ring_v, comb_v,
              sem_e, sem_o, sem_c, *sem_w):
    wid = lax.axis_index("s") * NC + lax.axis_index("c")
    base = wid * BPW
    iota = lax.iota(jnp.int32, 16)

    gc = pltpu.async_copy(comb_hbm, comb_v, sem_c)
    pltpu.sync_copy(p1_hbm.at[pl.ds(base, BPW + 16)], p1_v)
    pltpu.sync_copy(p2_hbm.at[pl.ds(base, BPW + 16)], p2_v)

    def fetch_lane(p, sem):
        fl = (p >> 17) & 1

        @pl.when(fl == 1)
        def _():
            off = pl.multiple_of((p & 0x1FFF) * 128, 128)
            rs = (p >> 13) & 15
            pltpu.async_copy(wT_hbm.at[:, pl.ds(off, 128)], ring_v.at[rs],
                             sem)

    def drain_lane(p, sem):
        fl = (p >> 17) & 1

        @pl.when(fl == 1)
        def _():
            pltpu.make_async_copy(wT_hbm.at[:, pl.ds(0, 128)],
                                  ring_v.at[0], sem).wait()

    def wave(v, _):
        p1v = p1_v[pl.ds(v * WV, 16)]
        p2v = p2_v[pl.ds(v * WV, 16)]

        for par in range(DEPTH):
            @pl.when(v % DEPTH == par)
            def _(par=par):
                @pl.when(v < NWAVE - DEPTH)
                def _():
                    for l in range(DEPTH * WV, (DEPTH + 1) * WV):
                        fetch_lane(p1v[l], sem_w[par])
                for l in range(WV):
                    drain_lane(p1v[l], sem_w[par])

        for l in range(WV):
            p = p1v[l]
            rs = (p >> 13) & 15
            col = (p >> 18) & 0x7F
            c = (p >> 25) & 3
            so = p2v[l]
            r = v * WV + l
            er = r & (HB - 1)
            for q in range(D // 16):
                sl = pl.ds(q * 16, 16)
                w = plsc.load_gather(
                    ring_v, [jnp.full((16,), rs, jnp.int32),
                             q * 16 + iota,
                             jnp.full((16,), col, jnp.int32)])
                e = eps_v[er, sl]
                m = comb_v[c, sl]
                s = comb_v[c, pl.ds(D + q * 16, 16)]
                eps_v[er, sl] = w * (m + s * e)
            pltpu.async_copy(eps_v.at[er], out_hbm.at[so], sem_o)
        return 0

    for h in range(BPW // HB):
        def eps_fire(g, _):
            ivec = p2_v[pl.ds(h * HB + g * 16, 16)]
            for j in range(16):
                pltpu.async_copy(eps_hbm.at[ivec[j]], eps_v.at[g * 16 + j],
                                 sem_e)
            return 0

        if h > 0:
            pltpu.make_async_copy(eps_hbm.at[pl.ds(0, HB)], eps_v,
                                  sem_o).wait()
        lax.fori_loop(0, HB // 16, eps_fire, 0)
        if h == 0:
            p1v0 = p1_v[pl.ds(0, 16)]
            for v0 in range(DEPTH):
                for l in range(WV):
                    fetch_lane(p1v0[v0 * WV + l], sem_w[v0 % DEPTH])
        pltpu.make_async_copy(eps_hbm.at[pl.ds(0, HB)], eps_v, sem_e).wait()
        if h == 0:
            gc.wait()
        lax.fori_loop(h * (HB // WV), (h + 1) * (HB // WV), wave, 0)

    pltpu.make_async_copy(eps_hbm.at[pl.ds(0, HB)], eps_v, sem_o).wait()


@jax.jit
def _sto(p1, p2, epsilon, weightT, comb):
    mesh = plsc.VectorSubcoreMesh(core_axis_name="c", subcore_axis_name="s")
    run = pl.kernel(
        _sto_body,
        out_type=jax.ShapeDtypeStruct((B, D), jnp.float32),
        mesh=mesh,
        compiler_params=pltpu.CompilerParams(
            use_tc_tiling_on_sc=True, disable_bounds_checks=True,
            needs_layout_passes=False),
        scratch_types=[
            pltpu.VMEM((BPW + 16,), jnp.int32),
            pltpu.VMEM((BPW + 16,), jnp.int32),
            pltpu.VMEM((HB, D), jnp.float32),
            pltpu.VMEM((RING, D, 128), jnp.float32),
            pltpu.VMEM((C, 2 * D), jnp.float32),
            pltpu.SemaphoreType.DMA,
            pltpu.SemaphoreType.DMA,
            pltpu.SemaphoreType.DMA,
        ] + [pltpu.SemaphoreType.DMA] * DEPTH,
    )
    return run(p1, p2, epsilon, weightT, comb)


def kernel(indices, comp_indices, epsilon, weight, post_mean, post_std):
    comb = jnp.concatenate(
        [post_mean, jax.nn.softplus(post_std)], axis=1).astype(jnp.float32)
    idx = indices.astype(jnp.int32)
    ci = comp_indices.astype(jnp.int32)
    blk = idx >> 7
    col = idx & 127
    orig = jnp.arange(B, dtype=jnp.int32)
    payload = (col << 18) | (ci << 16) | orig
    sblk, sp = lax.sort((blk, payload), num_keys=1)
    pos = jnp.arange(B, dtype=jnp.int32)
    first = jnp.concatenate(
        [jnp.ones((1,), jnp.int32),
         (sblk[1:] != sblk[:-1]).astype(jnp.int32)])
    flag = jnp.maximum(first, (pos % BPW == 0).astype(jnp.int32))
    rs = (jnp.cumsum(flag) - 1).astype(jnp.int32) % RING
    p1 = (sblk | (rs << 13) | (flag << 17) | (sp & (0x7F << 18))
          | (((sp >> 16) & 3) << 25))
    p2 = sp & 0x3FFF
    p1 = jnp.pad(p1, (0, 16))
    p2 = jnp.pad(p2, (0, 16))
    return _sto(p1, p2, epsilon, weight.T, comb)


# speedup vs baseline: 5.7702x; 1.0386x over previous
import jax
import jax.numpy as jnp
from jax import lax
from jax.experimental import pallas as pl
from jax.experimental.pallas import tpu as pltpu
from jax.experimental.pallas import tpu_sc as plsc

B = 16384
V = 1000000
D = 64
C = 4

NC = 2
NS = 16
NW = NC * NS
BPW = B // NW
WV = 2
NWAVE = BPW // WV
DEPTH = 5
RING = 13
HB = 128


def _sto_body(p1_hbm, p2_hbm, eps_hbm, wT_hbm, comb_hbm, out_hbm,
              p1_v, p2_v, eps_v, ring_v, comb_v,
              sem_e, sem_o, sem_c, *sem_w):
    wid = lax.axis_index("s") * NC + lax.axis_index("c")
    base = wid * BPW
    iota = lax.iota(jnp.int32, 16)

    gc = pltpu.async_copy(comb_hbm, comb_v, sem_c)
    pltpu.sync_copy(p1_hbm.at[pl.ds(base, BPW + 16)], p1_v)
    pltpu.sync_copy(p2_hbm.at[pl.ds(base, BPW + 16)], p2_v)

    def fetch_lane(p, sem):
        fl = (p >> 17) & 1

        @pl.when(fl == 1)
        def _():
            off = pl.multiple_of((p & 0x1FFF) * 128, 128)
            rs = (p >> 13) & 15
            pltpu.async_copy(wT_hbm.at[:, pl.ds(off, 128)], ring_v.at[rs],
                             sem)

    def drain_lane(p, sem):
        fl = (p >> 17) & 1

        @pl.when(fl == 1)
        def _():
            pltpu.make_async_copy(wT_hbm.at[:, pl.ds(0, 128)],
                                  ring_v.at[0], sem).wait()

    def wave(v, _):
        p1v = p1_v[pl.ds(v * WV, 16)]
        p2v = p2_v[pl.ds(v * WV, 16)]

        for par in range(DEPTH):
            @pl.when(v % DEPTH == par)
            def _(par=par):
                @pl.when(v < NWAVE - DEPTH)
                def _():
                    for l in range(DEPTH * WV, (DEPTH + 1) * WV):
                        fetch_lane(p1v[l], sem_w[par])
                for l in range(WV):
                    drain_lane(p1v[l], sem_w[par])

        for l in range(WV):
            p = p1v[l]
            rs = (p >> 13) & 15
            col = (p >> 18) & 0x7F
            c = (p >> 25) & 3
            so = p2v[l]
            r = v * WV + l
            er = r & (HB - 1)
            for q in range(D // 16):
                sl = pl.ds(q * 16, 16)
                w = plsc.load_gather(
                    ring_v, [jnp.full((16,), rs, jnp.int32),
                             q * 16 + iota,
                             jnp.full((16,), col, jnp.int32)])
                e = eps_v[er, sl]
                m = comb_v[c, sl]
                s = comb_v[c, pl.ds(D + q * 16, 16)]
                eps_v[er, sl] = w * (m + s * e)
            pltpu.async_copy(eps_v.at[er], out_hbm.at[so], sem_o)
        return 0

    for h in range(BPW // HB):
        def eps_fire(g, _):
            ivec = p2_v[pl.ds(h * HB + g * 16, 16)]
            for j in range(16):
                pltpu.async_copy(eps_hbm.at[ivec[j]], eps_v.at[g * 16 + j],
                                 sem_e)
            return 0

        if h > 0:
            pltpu.make_async_copy(eps_hbm.at[pl.ds(0, HB)], eps_v,
                                  sem_o).wait()
        lax.fori_loop(0, HB // 16, eps_fire, 0)
        if h == 0:
            p1v0 = p1_v[pl.ds(0, 16)]
            for v0 in range(DEPTH):
                for l in range(WV):
                    fetch_lane(p1v0[v0 * WV + l], sem_w[v0 % DEPTH])
        pltpu.make_async_copy(eps_hbm.at[pl.ds(0, HB)], eps_v, sem_e).wait()
        if h == 0:
            gc.wait()
        lax.fori_loop(h * (HB // WV), (h + 1) * (HB // WV), wave, 0)

    pltpu.make_async_copy(eps_hbm.at[pl.ds(0, HB)], eps_v, sem_o).wait()


@jax.jit
def _sto(p1, p2, epsilon, weightT, comb):
    mesh = plsc.VectorSubcoreMesh(core_axis_name="c", subcore_axis_name="s")
    run = pl.kernel(
        _sto_body,
        out_type=jax.ShapeDtypeStruct((B, D), jnp.float32),
        mesh=mesh,
        compiler_params=pltpu.CompilerParams(
            use_tc_tiling_on_sc=True, disable_bounds_checks=True,
            needs_layout_passes=False),
        scratch_types=[
            pltpu.VMEM((BPW + 16,), jnp.int32),
            pltpu.VMEM((BPW + 16,), jnp.int32),
            pltpu.VMEM((HB, D), jnp.float32),
            pltpu.VMEM((RING, D, 128), jnp.float32),
            pltpu.VMEM((C, 2 * D), jnp.float32),
            pltpu.SemaphoreType.DMA,
            pltpu.SemaphoreType.DMA,
            pltpu.SemaphoreType.DMA,
        ] + [pltpu.SemaphoreType.DMA] * DEPTH,
    )
    return run(p1, p2, epsilon, weightT, comb)


def kernel(indices, comp_indices, epsilon, weight, post_mean, post_std):
    comb = jnp.concatenate(
        [post_mean, jax.nn.softplus(post_std)], axis=1).astype(jnp.float32)
    idx = indices.astype(jnp.int32)
    ci = comp_indices.astype(jnp.int32)
    blk = idx >> 7
    col = idx & 127
    orig = jnp.arange(B, dtype=jnp.int32)
    payload = (col << 18) | (ci << 16) | orig
    sblk, sp = lax.sort((blk, payload), num_keys=1)
    pos = jnp.arange(B, dtype=jnp.int32)
    first = jnp.concatenate(
        [jnp.ones((1,), jnp.int32),
         (sblk[1:] != sblk[:-1]).astype(jnp.int32)])
    flag = jnp.maximum(first, (pos % BPW == 0).astype(jnp.int32))
    rs = (jnp.cumsum(flag) - 1).astype(jnp.int32) % RING
    p1 = (sblk | (rs << 13) | (flag << 17) | (sp & (0x7F << 18))
          | (((sp >> 16) & 3) << 25))
    p2 = sp & 0x3FFF
    p1 = jnp.pad(p1, (0, 16))
    p2 = jnp.pad(p2, (0, 16))
    return _sto(p1, p2, epsilon, weight.T, comb)
